# baseline (speedup 1.0000x reference)
import dataclasses
import functools

import jax
import jax.numpy as jnp
from jax import lax
from jax.experimental import pallas as pl
from jax.experimental.pallas import tpu as pltpu
from jax.experimental.pallas import tpu_sc as plsc

N, E, D, H, O, B = 10000, 320000, 128, 128, 128, 64

NC, NS = 2, 16
NW = NC * NS
CH = 128
NCHUNK = 84
HALF = NCHUNK // 2
EP = NW * NCHUNK * CH
NP = 10240
ZT = 10
ZROWS = N // ZT


NR = NP // 128
NRD = NR // NS


def _deg_body(dst_hbm, out_hbm, idx_v, hist_v, iota_v, acc_sh):
    cid = lax.axis_index("c")
    sid = lax.axis_index("s")
    wid = cid * NS + sid

    @pl.loop(0, NR)
    def _(r):
        @pl.loop(0, 8)
        def _(k):
            hist_v[r, pl.ds(k * 16, 16)] = jnp.zeros((16,), jnp.float32)

    @pl.loop(0, NR // 16)
    def _(g):
        iota_v[g, :] = lax.iota(jnp.int32, 16) + g * 16

    @pl.when(sid == 0)
    def _():
        pltpu.sync_copy(hist_v, acc_sh)
    plsc.subcore_barrier()

    pltpu.sync_copy(dst_hbm.at[wid], idx_v)
    ones = jnp.ones((16,), jnp.float32)

    @pl.loop(0, NCHUNK)
    def _(j):
        @pl.loop(0, CH // 16)
        def _(g):
            iv = idx_v[j, pl.ds(g * 16, 16)]
            plsc.addupdate_scatter(hist_v, [iv >> 7, iv & 127], ones)

    @pl.loop(0, NR // 16)
    def _(g):
        pltpu.sync_copy(hist_v.at[pl.ds(g * 16, 16)],
                        acc_sh.at[iota_v.at[g]], add=True)

    plsc.subcore_barrier()

    @pl.when(sid == 0)
    def _():
        pltpu.sync_copy(acc_sh, hist_v)
        pltpu.sync_copy(hist_v, out_hbm.at[cid])


_ZPIECES = [(128, i * 128) for i in range(7)] + [(104, 896)]


def _msg_body(xs_hbm, src_hbm, dst_hbm, out_hbm, src_v, dst_v, bufa, bufb,
              acc_sh, sema, semb):
    cid = lax.axis_index("c")
    sid = lax.axis_index("s")
    wid = cid * NS + sid

    @pl.loop(0, CH)
    def _(r):
        @pl.loop(0, D // 16)
        def _(k):
            bufa[r, pl.ds(k * 16, 16)] = jnp.zeros((16,), jnp.float32)

    @pl.when(sid < ZT)
    def _():
        for sz, off in _ZPIECES:
            pltpu.sync_copy(bufa.at[pl.ds(0, sz)],
                            acc_sh.at[pl.ds(sid * ZROWS + off, sz)])

    plsc.subcore_barrier()

    @pl.loop(0, 2)
    def _(h):
        pltpu.sync_copy(src_hbm.at[wid, h], src_v)
        pltpu.sync_copy(dst_hbm.at[wid, h], dst_v)
        pltpu.async_copy(xs_hbm.at[src_v.at[0]], bufa, sema)

        @pl.loop(0, HALF // 2)
        def _(p):
            j = p * 2
            pltpu.make_async_copy(xs_hbm.at[src_v.at[j]], bufa, sema).wait()
            pltpu.async_copy(xs_hbm.at[src_v.at[j + 1]], bufb, semb)
            pltpu.sync_copy(bufa, acc_sh.at[dst_v.at[j]], add=True)
            pltpu.make_async_copy(xs_hbm.at[src_v.at[j + 1]], bufb, semb).wait()

            @pl.when(p + 1 < HALF // 2)
            def _():
                pltpu.async_copy(xs_hbm.at[src_v.at[j + 2]], bufa, sema)

            pltpu.sync_copy(bufb, acc_sh.at[dst_v.at[j + 1]], add=True)

    plsc.subcore_barrier()

    @pl.when(sid < ZT)
    def _():
        for sz, off in _ZPIECES:
            pltpu.sync_copy(acc_sh.at[pl.ds(sid * ZROWS + off, sz)],
                            bufa.at[pl.ds(0, sz)])
            pltpu.sync_copy(bufa.at[pl.ds(0, sz)],
                            out_hbm.at[cid, pl.ds(sid * ZROWS + off, sz)])


@functools.cache
def _sc_kernels():
    mesh = plsc.VectorSubcoreMesh(core_axis_name="c", subcore_axis_name="s",
                                  num_cores=NC, num_subcores=NS)
    cp = pltpu.CompilerParams()
    if "needs_layout_passes" in pltpu.CompilerParams.__dataclass_fields__:
        cp = dataclasses.replace(cp, needs_layout_passes=False)
    deg_call = pl.kernel(
        _deg_body,
        out_type=jax.ShapeDtypeStruct((NC, NR, 128), jnp.float32),
        mesh=mesh,
        compiler_params=cp,
        scratch_types=[
            pltpu.VMEM((NCHUNK, CH), jnp.int32),
            pltpu.VMEM((NR, 128), jnp.float32),
            pltpu.VMEM((NR // 16, 16), jnp.int32),
            pltpu.VMEM_SHARED((NR, 128), jnp.float32),
        ],
    )
    msg_call = pl.kernel(
        _msg_body,
        out_type=jax.ShapeDtypeStruct((NC, N, D), jnp.float32),
        mesh=mesh,
        scratch_types=[
            pltpu.VMEM((HALF, CH), jnp.int32),
            pltpu.VMEM((HALF, CH), jnp.int32),
            pltpu.VMEM((CH, D), jnp.float32),
            pltpu.VMEM((CH, D), jnp.float32),
            pltpu.VMEM_SHARED((N, D), jnp.float32),
            pltpu.SemaphoreType.DMA,
            pltpu.SemaphoreType.DMA,
        ],
    )
    return deg_call, msg_call


def _dinv_from_parts(degp):
    deg = degp[0] + degp[1]
    return jnp.where(deg > 0, lax.rsqrt(deg), 0.0)


def _tc_prescale_body(x_ref, w_ref, degp_ref, o_ref):
    dinv = _dinv_from_parts(degp_ref[...])
    xw = jnp.dot(x_ref[...], w_ref[...], preferred_element_type=jnp.float32)
    o_ref[...] = xw * dinv


def _tc_mid_body(parts_ref, degp_ref, b_ref, w_ref, o_ref):
    dinv = _dinv_from_parts(degp_ref[...])[:N]
    agg = parts_ref[0] + parts_ref[1]
    h = jnp.maximum(agg * dinv + b_ref[...], 0.0)
    o_ref[0:N, :] = jnp.dot(h, w_ref[...],
                            preferred_element_type=jnp.float32) * dinv
    o_ref[N:NP, :] = jnp.zeros((NP - N, D), jnp.float32)


def _tc_final_body(parts_ref, degp_ref, b_ref, batch_ref, lw_ref, lb_ref, o_ref):
    dinv = _dinv_from_parts(degp_ref[...])[:N]
    agg = parts_ref[0] + parts_ref[1]
    h = jnp.maximum(agg * dinv + b_ref[...], 0.0)
    seg = lax.broadcasted_iota(jnp.int32, (B, N), 0)
    onehot = jnp.where(seg == jnp.broadcast_to(batch_ref[...], (B, N)), 1.0, 0.0)
    sums = jnp.dot(onehot, h, preferred_element_type=jnp.float32)
    counts = jnp.sum(onehot, axis=1, keepdims=True)
    pooled = sums / jnp.maximum(counts, 1.0)
    o_ref[...] = (jnp.dot(pooled, lw_ref[...], preferred_element_type=jnp.float32)
                  + lb_ref[...])


_prescale = pl.pallas_call(
    _tc_prescale_body, out_shape=jax.ShapeDtypeStruct((NP, D), jnp.float32))
_mid = pl.pallas_call(
    _tc_mid_body, out_shape=jax.ShapeDtypeStruct((NP, D), jnp.float32))
_final = pl.pallas_call(
    _tc_final_body, out_shape=jax.ShapeDtypeStruct((B, O), jnp.float32))


def kernel(x, edge_index, batch, W1, b1, W2, b2, lin_W, lin_b):
    loop = jnp.arange(N, dtype=jnp.int32)
    npad = EP - E - N
    src = jnp.concatenate([edge_index[0], loop, jnp.full((npad,), N, jnp.int32)]
                          ).reshape(NW, 2, HALF, CH)
    dst_m = jnp.concatenate([edge_index[1], loop, jnp.zeros((npad,), jnp.int32)]
                            ).reshape(NW, 2, HALF, CH)
    dst_d = jnp.concatenate([edge_index[1], loop, jnp.full((npad,), N, jnp.int32)]
                            ).reshape(NW, NCHUNK, CH)
    xp = jnp.pad(x, ((0, NP - N), (0, 0)))
    batch_r = batch.reshape(1, N)

    deg_call, msg_call = _sc_kernels()
    degp = deg_call(dst_d).reshape(NC, NP, 1)
    xs1 = _prescale(xp, W1, degp)
    parts1 = msg_call(xs1, src, dst_m)
    xs2 = _mid(parts1, degp, b1, W2)
    parts2 = msg_call(xs2, src, dst_m)
    return _final(parts2, degp, b2, batch_r, lin_W, lin_b)


# speedup vs baseline: 4.6581x; 4.6581x over previous
import dataclasses
import functools

import jax
import jax.numpy as jnp
from jax import lax
from jax.experimental import pallas as pl
from jax.experimental.pallas import tpu as pltpu
from jax.experimental.pallas import tpu_sc as plsc

N, E, D, H, O, B = 10000, 320000, 128, 128, 128, 64

NC, NS = 2, 16
NW = NC * NS
CH = 128
NCHUNK = 84
HALF = NCHUNK // 2
EP = NW * NCHUNK * CH
NP = 10240
ZT = 10
ZROWS = N // ZT


NR = NP // 128
NRD = NR // NS


def _deg_body(dst_hbm, out_hbm, idx_v, hist_v, iota_v, acc_sh):
    cid = lax.axis_index("c")
    sid = lax.axis_index("s")
    wid = cid * NS + sid

    @pl.loop(0, NR)
    def _(r):
        @pl.loop(0, 8)
        def _(k):
            hist_v[r, pl.ds(k * 16, 16)] = jnp.zeros((16,), jnp.float32)

    @pl.loop(0, NR // 16)
    def _(g):
        iota_v[g, :] = lax.iota(jnp.int32, 16) + g * 16

    @pl.when(sid == 0)
    def _():
        pltpu.sync_copy(hist_v, acc_sh)
    plsc.subcore_barrier()

    pltpu.sync_copy(dst_hbm.at[wid], idx_v)
    ones = jnp.ones((16,), jnp.float32)

    @pl.loop(0, NCHUNK)
    def _(j):
        @pl.loop(0, CH // 16)
        def _(g):
            iv = idx_v[j, pl.ds(g * 16, 16)]
            plsc.addupdate_scatter(hist_v, [iv >> 7, iv & 127], ones)

    @pl.loop(0, NR // 16)
    def _(g):
        pltpu.sync_copy(hist_v.at[pl.ds(g * 16, 16)],
                        acc_sh.at[iota_v.at[g]], add=True)

    plsc.subcore_barrier()

    @pl.when(sid == 0)
    def _():
        pltpu.sync_copy(acc_sh, hist_v)
        pltpu.sync_copy(hist_v, out_hbm.at[cid])


_ZPIECES = [(128, i * 128) for i in range(7)] + [(104, 896)]


def _msg_body(xs_hbm, src_hbm, dst_hbm, out_hbm, src_v, dst_v, bufa, bufb,
              acc_sh, sema, semb):
    cid = lax.axis_index("c")
    sid = lax.axis_index("s")
    wid = cid * NS + sid

    @pl.loop(0, CH)
    def _(r):
        @pl.loop(0, D // 16)
        def _(k):
            bufa[r, pl.ds(k * 16, 16)] = jnp.zeros((16,), jnp.float32)

    @pl.when(sid < ZT)
    def _():
        for sz, off in _ZPIECES:
            pltpu.sync_copy(bufa.at[pl.ds(0, sz)],
                            acc_sh.at[pl.ds(sid * ZROWS + off, sz)])

    plsc.subcore_barrier()

    @pl.loop(0, 2)
    def _(h):
        pltpu.sync_copy(src_hbm.at[wid, h], src_v)
        pltpu.sync_copy(dst_hbm.at[wid, h], dst_v)
        pltpu.async_copy(xs_hbm.at[src_v.at[0]], bufa, sema)

        @pl.loop(0, HALF // 2)
        def _(p):
            j = p * 2
            pltpu.make_async_copy(xs_hbm.at[src_v.at[j]], bufa, sema).wait()
            pltpu.async_copy(xs_hbm.at[src_v.at[j + 1]], bufb, semb)
            pltpu.sync_copy(bufa, acc_sh.at[dst_v.at[j]], add=True)
            pltpu.make_async_copy(xs_hbm.at[src_v.at[j + 1]], bufb, semb).wait()

            @pl.when(p + 1 < HALF // 2)
            def _():
                pltpu.async_copy(xs_hbm.at[src_v.at[j + 2]], bufa, sema)

            pltpu.sync_copy(bufb, acc_sh.at[dst_v.at[j + 1]], add=True)

    plsc.subcore_barrier()

    @pl.when(sid < ZT)
    def _():
        for sz, off in _ZPIECES:
            pltpu.sync_copy(acc_sh.at[pl.ds(sid * ZROWS + off, sz)],
                            bufa.at[pl.ds(0, sz)])
            pltpu.sync_copy(bufa.at[pl.ds(0, sz)],
                            out_hbm.at[cid, pl.ds(sid * ZROWS + off, sz)])


@functools.cache
def _sc_kernels():
    mesh = plsc.VectorSubcoreMesh(core_axis_name="c", subcore_axis_name="s",
                                  num_cores=NC, num_subcores=NS)
    cp = pltpu.CompilerParams()
    if "needs_layout_passes" in pltpu.CompilerParams.__dataclass_fields__:
        cp = dataclasses.replace(cp, needs_layout_passes=False)
    deg_call = pl.kernel(
        _deg_body,
        out_type=jax.ShapeDtypeStruct((NC, NR, 128), jnp.float32),
        mesh=mesh,
        compiler_params=cp,
        scratch_types=[
            pltpu.VMEM((NCHUNK, CH), jnp.int32),
            pltpu.VMEM((NR, 128), jnp.float32),
            pltpu.VMEM((NR // 16, 16), jnp.int32),
            pltpu.VMEM_SHARED((NR, 128), jnp.float32),
        ],
    )
    msg_call = pl.kernel(
        _msg_body,
        out_type=jax.ShapeDtypeStruct((NC, N, D), jnp.float32),
        mesh=mesh,
        scratch_types=[
            pltpu.VMEM((HALF, CH), jnp.int32),
            pltpu.VMEM((HALF, CH), jnp.int32),
            pltpu.VMEM((CH, D), jnp.float32),
            pltpu.VMEM((CH, D), jnp.float32),
            pltpu.VMEM_SHARED((N, D), jnp.float32),
            pltpu.SemaphoreType.DMA,
            pltpu.SemaphoreType.DMA,
        ],
    )
    return deg_call, msg_call


def _dinv_from_parts(degp):
    deg = degp[0] + degp[1]
    return jnp.where(deg > 0, lax.rsqrt(deg), 0.0)


def _tc_prescale_body(x_ref, w_ref, degp_ref, o_ref):
    dinv = _dinv_from_parts(degp_ref[...])
    xw = jnp.dot(x_ref[...], w_ref[...], preferred_element_type=jnp.float32)
    o_ref[...] = xw * dinv


def _tc_mid_body(parts_ref, degp_ref, b_ref, w_ref, o_ref):
    dinv = _dinv_from_parts(degp_ref[...])[:N]
    agg = parts_ref[0] + parts_ref[1]
    h = jnp.maximum(agg * dinv + b_ref[...], 0.0)
    o_ref[0:N, :] = jnp.dot(h, w_ref[...],
                            preferred_element_type=jnp.float32) * dinv
    o_ref[N:NP, :] = jnp.zeros((NP - N, D), jnp.float32)


def _tc_final_body(parts_ref, degp_ref, b_ref, batch_ref, lw_ref, lb_ref, o_ref):
    dinv = _dinv_from_parts(degp_ref[...])[:N]
    agg = parts_ref[0] + parts_ref[1]
    h = jnp.maximum(agg * dinv + b_ref[...], 0.0)
    seg = lax.broadcasted_iota(jnp.int32, (B, N), 0)
    onehot = jnp.where(seg == jnp.broadcast_to(batch_ref[...], (B, N)), 1.0, 0.0)
    sums = jnp.dot(onehot, h, preferred_element_type=jnp.float32)
    counts = jnp.sum(onehot, axis=1, keepdims=True)
    pooled = sums / jnp.maximum(counts, 1.0)
    o_ref[...] = (jnp.dot(pooled, lw_ref[...], preferred_element_type=jnp.float32)
                  + lb_ref[...])


_prescale = pl.pallas_call(
    _tc_prescale_body, out_shape=jax.ShapeDtypeStruct((NP, D), jnp.float32))
_mid = pl.pallas_call(
    _tc_mid_body, out_shape=jax.ShapeDtypeStruct((NP, D), jnp.float32))
_final = pl.pallas_call(
    _tc_final_body, out_shape=jax.ShapeDtypeStruct((B, O), jnp.float32))


def kernel(x, edge_index, batch, W1, b1, W2, b2, lin_W, lin_b):
    loop = jnp.arange(N, dtype=jnp.int32)
    npad = EP - E - N
    ppos = jnp.arange(npad, dtype=jnp.int32)
    src_pad = N + ppos % (NP - N)
    src = jnp.concatenate([edge_index[0], loop, src_pad]
                          ).reshape(NW, 2, HALF, CH)
    dst_m = jnp.concatenate([edge_index[1], loop, ppos % N]
                            ).reshape(NW, 2, HALF, CH)
    dst_d = jnp.concatenate([edge_index[1], loop, src_pad]
                            ).reshape(NW, NCHUNK, CH)
    xp = jnp.pad(x, ((0, NP - N), (0, 0)))
    batch_r = batch.reshape(1, N)

    deg_call, msg_call = _sc_kernels()
    degp = deg_call(dst_d).reshape(NC, NP, 1)
    xs1 = _prescale(xp, W1, degp)
    parts1 = msg_call(xs1, src, dst_m)
    xs2 = _mid(parts1, degp, b1, W2)
    parts2 = msg_call(xs2, src, dst_m)
    return _final(parts2, degp, b2, batch_r, lin_W, lin_b)
